# Initial kernel scaffold; baseline (speedup 1.0000x reference)
import jax
import jax.numpy as jnp
from jax.experimental import pallas as pl


def kernel(x, edge_index, W, b):
    raise NotImplementedError("write your pallas kernel here")


# speedup vs baseline: 4.3886x; 4.3886x over previous
import functools

import jax
import jax.numpy as jnp
from jax import lax
from jax.experimental import pallas as pl
from jax.experimental.pallas import tpu as pltpu
from jax.experimental.pallas import tpu_sc as plsc

NN = 10000
EE = 160000
DD = 256
NC = 2
NS = 16
LL = 16
HALF = DD // NC
BATCH = 128
NB = EE // BATCH
NBF = NB // NS
NBT = NB - NBF * NS
RP = 10240
GARB = NN
RPT = RP // NS
CHUNKS = RPT // BATCH

_mesh = plsc.VectorSubcoreMesh(core_axis_name="c", subcore_axis_name="s")


def _edge_loop(wid, per_batch):
    @pl.loop(0, NBF)
    def _(ii):
        per_batch(ii * NS + wid)

    @pl.when(wid < NBT)
    def _():
        per_batch(NBF * NS + wid)


def _sc_degrees(ei, ones16, z16):
    @functools.partial(
        pl.kernel,
        out_type=[
            jax.ShapeDtypeStruct((NC * 3 * RP, LL), jnp.float32),
            jax.ShapeDtypeStruct((NC * 4 * RP, LL), jnp.float32),
        ],
        mesh=_mesh,
        compiler_params=pltpu.CompilerParams(use_tc_tiling_on_sc=False),
        scratch_types=[
            pltpu.VMEM_SHARED((4 * RP, LL), jnp.float32),
            pltpu.VMEM((BATCH, LL), jnp.float32),
            pltpu.VMEM((BATCH, LL), jnp.float32),
            pltpu.VMEM((BATCH, LL), jnp.float32),
            pltpu.VMEM((1, BATCH), jnp.int32),
            pltpu.VMEM((1, BATCH), jnp.int32),
            pltpu.VMEM((1, BATCH), jnp.int32),
            pltpu.VMEM((1, BATCH), jnp.int32),
        ],
    )
    def k(ei_hbm, ones_hbm, z_hbm, oh_hbm, os_hbm, slab, obuf, zbuf, rows,
          rbuf, cbuf, gidx, sidx):
        cid = lax.axis_index("c")
        wid = lax.axis_index("s")
        pltpu.sync_copy(ones_hbm, obuf)
        pltpu.sync_copy(z_hbm, zbuf)

        def zero_slab(nregions):
            nchunks = nregions * RP // NS // BATCH

            @pl.loop(0, nchunks)
            def _(i):
                pltpu.sync_copy(
                    zbuf, slab.at[pl.ds(wid * (nchunks * BATCH) + i * BATCH,
                                        BATCH)])

        def load_edges(bi):
            e0 = bi * BATCH
            pltpu.sync_copy(ei_hbm.at[pl.ds(0, 1), pl.ds(e0, BATCH)], rbuf)
            pltpu.sync_copy(ei_hbm.at[pl.ds(1, 1), pl.ds(e0, BATCH)], cbuf)

        zero_slab(3)
        plsc.subcore_barrier()

        def hist_batch(bi):
            load_edges(bi)
            pltpu.sync_copy(obuf, slab.at[rbuf.at[0]], add=True)
            for kk in range(BATCH // LL):
                sl = pl.ds(kk * LL, LL)
                sidx[0, sl] = cbuf[0, sl] + RP
            pltpu.sync_copy(obuf, slab.at[sidx.at[0]], add=True)
            for kk in range(BATCH // LL):
                sl = pl.ds(kk * LL, LL)
                rv = rbuf[0, sl]
                cv = cbuf[0, sl]
                sidx[0, sl] = jnp.where(rv == cv, rv, GARB) + 2 * RP
            pltpu.sync_copy(obuf, slab.at[sidx.at[0]], add=True)

        _edge_loop(wid, hist_batch)
        plsc.subcore_barrier()

        hchunks = 3 * RP // NS // BATCH

        @pl.loop(0, hchunks)
        def _(i):
            ro = wid * (hchunks * BATCH) + i * BATCH
            pltpu.sync_copy(slab.at[pl.ds(ro, BATCH)], rows)
            pltpu.sync_copy(rows, oh_hbm.at[pl.ds(cid * 3 * RP + ro, BATCH)])

        plsc.subcore_barrier()

        zero_slab(4)
        plsc.subcore_barrier()

        spmv_spec = [(True, 0, False), (False, 1, True),
                     (True, 1, False), (False, 0, True)]

        def spmv_batch(bi):
            load_edges(bi)
            hbase = cid * 3 * RP
            for j, (g_is_c, greg, s_is_c) in enumerate(spmv_spec):
                for kk in range(BATCH // LL):
                    sl = pl.ds(kk * LL, LL)
                    gv = cbuf[0, sl] if g_is_c else rbuf[0, sl]
                    sv = cbuf[0, sl] if s_is_c else rbuf[0, sl]
                    gidx[0, sl] = gv + (hbase + greg * RP)
                    sidx[0, sl] = sv + j * RP
                pltpu.sync_copy(oh_hbm.at[gidx.at[0]], rows)
                pltpu.sync_copy(rows, slab.at[sidx.at[0]], add=True)

        _edge_loop(wid, spmv_batch)
        plsc.subcore_barrier()

        schunks = 4 * RP // NS // BATCH

        @pl.loop(0, schunks)
        def _(i):
            ro = wid * (schunks * BATCH) + i * BATCH
            pltpu.sync_copy(slab.at[pl.ds(ro, BATCH)], rows)
            pltpu.sync_copy(rows, os_hbm.at[pl.ds(cid * 4 * RP + ro, BATCH)])

    return k(ei, ones16, z16)


def _sc_prop(ei, ytab, z128):
    s1_spec = [(True, True, True), (False, True, True),
               (True, False, False), (False, False, False),
               (True, False, False), (False, False, False)]
    s2_spec = [(2, True), (3, True), (4, False), (5, False)]

    @functools.partial(
        pl.kernel,
        out_type=[
            jax.ShapeDtypeStruct((6 * NC * RP, HALF), jnp.float32),
            jax.ShapeDtypeStruct((4 * NC * RP, HALF), jnp.float32),
        ],
        mesh=_mesh,
        scratch_types=[
            pltpu.VMEM_SHARED((RP, HALF), jnp.float32),
            pltpu.VMEM((BATCH, HALF), jnp.float32),
            pltpu.VMEM((BATCH, HALF), jnp.float32),
            pltpu.VMEM((1, BATCH), jnp.int32),
            pltpu.VMEM((1, BATCH), jnp.int32),
            pltpu.VMEM((1, BATCH), jnp.int32),
            pltpu.VMEM((1, BATCH), jnp.int32),
        ],
    )
    def k(ei_hbm, y_hbm, z_hbm, o1_hbm, o2_hbm, slab, rows, zbuf,
          rbuf, cbuf, gidx, sidx):
        cid = lax.axis_index("c")
        wid = lax.axis_index("s")
        tb = wid * RPT
        pltpu.sync_copy(z_hbm, zbuf)

        def run_stream(table_hbm, tbase, out_hbm, obase, src_is_c, selfg,
                       inity):
            if inity:
                @pl.loop(0, CHUNKS)
                def _(i):
                    ro = tb + i * BATCH
                    pltpu.sync_copy(table_hbm.at[pl.ds(tbase + ro, BATCH)],
                                    rows)
                    pltpu.sync_copy(rows, slab.at[pl.ds(ro, BATCH)])
            else:
                @pl.loop(0, CHUNKS)
                def _(i):
                    pltpu.sync_copy(zbuf, slab.at[pl.ds(tb + i * BATCH,
                                                        BATCH)])
            plsc.subcore_barrier()

            def edge_batch(bi):
                e0 = bi * BATCH
                pltpu.sync_copy(ei_hbm.at[pl.ds(0, 1), pl.ds(e0, BATCH)],
                                rbuf)
                pltpu.sync_copy(ei_hbm.at[pl.ds(1, 1), pl.ds(e0, BATCH)],
                                cbuf)
                for kk in range(BATCH // LL):
                    sl = pl.ds(kk * LL, LL)
                    rv = rbuf[0, sl]
                    cv = cbuf[0, sl]
                    sv = cv if src_is_c else rv
                    dv = rv if src_is_c else cv
                    gidx[0, sl] = sv + tbase
                    if selfg:
                        dv = jnp.where(rv == cv, GARB, dv)
                    sidx[0, sl] = dv
                pltpu.sync_copy(table_hbm.at[gidx.at[0]], rows)
                pltpu.sync_copy(rows, slab.at[sidx.at[0]], add=True)

            _edge_loop(wid, edge_batch)
            plsc.subcore_barrier()

            @pl.loop(0, CHUNKS)
            def _(i):
                ro = tb + i * BATCH
                pltpu.sync_copy(slab.at[pl.ds(ro, BATCH)], rows)
                pltpu.sync_copy(rows, out_hbm.at[pl.ds(obase + ro, BATCH)])

            plsc.subcore_barrier()

        for s, (src_c, selfg, inity) in enumerate(s1_spec):
            rbase = s * NC * RP + cid * RP
            run_stream(y_hbm, rbase, o1_hbm, rbase, src_c, selfg, inity)
        for j, (sreg, src_c) in enumerate(s2_spec):
            tbase = sreg * NC * RP + cid * RP
            obase = j * NC * RP + cid * RP
            run_stream(o1_hbm, tbase, o2_hbm, obase, src_c, False, False)

    return k(ei, ytab, z128)


def _tc_matmul(x, W, inin):
    TN = 400

    def body(x_ref, w_ref, s_ref, y_ref):
        i = pl.program_id(0)
        acc = lax.dot_general(
            x_ref[...], w_ref[0],
            dimension_numbers=(((1,), (0,)), ((), ())),
            preferred_element_type=jnp.float32,
            precision=lax.Precision.HIGHEST)
        onehot = (lax.broadcasted_iota(jnp.int32, (1, 6), 1) == i)
        scale = jnp.sum(jnp.where(onehot, s_ref[...], 0.0), axis=1,
                        keepdims=True)
        y_ref[0] = acc * scale

    return pl.pallas_call(
        body,
        grid=(6, NC, NN // TN),
        in_specs=[
            pl.BlockSpec((TN, DD), lambda i, c, t: (t, 0)),
            pl.BlockSpec((1, DD, HALF), lambda i, c, t: (i, 0, c)),
            pl.BlockSpec((TN, 6), lambda i, c, t: (t, 0)),
        ],
        out_specs=pl.BlockSpec((1, TN, HALF), lambda i, c, t: (i * NC + c, t, 0)),
        out_shape=jax.ShapeDtypeStruct((6 * NC, RP, HALF), jnp.float32),
    )(x, W, inin)


def _tc_final(o1, o2, outv, bsum):
    TN = 400

    def body(a_ref, v_ref, s_ref, b_ref, t_ref):
        acc = jnp.zeros((TN, DD), jnp.float32) + b_ref[0][None, :]
        sc = s_ref[...]
        for i in range(2):
            h = jnp.concatenate([a_ref[2 * i], a_ref[2 * i + 1]], axis=1)
            acc = acc + h * sc[:, i:i + 1]
        for j in range(4):
            h = jnp.concatenate([v_ref[2 * j], v_ref[2 * j + 1]], axis=1)
            acc = acc + h * sc[:, 2 + j:3 + j]
        t_ref[...] = acc

    return pl.pallas_call(
        body,
        grid=(NN // TN,),
        in_specs=[
            pl.BlockSpec((4, TN, HALF), lambda t: (0, t, 0)),
            pl.BlockSpec((8, TN, HALF), lambda t: (0, t, 0)),
            pl.BlockSpec((TN, 6), lambda t: (t, 0)),
            pl.BlockSpec((1, DD), lambda t: (0, 0)),
        ],
        out_specs=pl.BlockSpec((TN, DD), lambda t: (t, 0)),
        out_shape=jax.ShapeDtypeStruct((NN, DD), jnp.float32),
    )(o1, o2, outv, bsum)


def kernel(x, edge_index, W, b):
    ei = edge_index.astype(jnp.int32)
    ones16 = jnp.ones((BATCH, LL), jnp.float32)
    z16 = jnp.zeros((BATCH, LL), jnp.float32)
    z128 = jnp.zeros((BATCH, HALF), jnp.float32)

    oh, osv = _sc_degrees(ei, ones16, z16)
    ohr = oh.reshape(NC, 3, RP, LL)
    rowdeg = ohr[0, 0, :NN, 0]
    coldeg = ohr[0, 1, :NN, 0]
    selfdeg = ohr[0, 2, :NN, 0]
    osr = osv.reshape(NC, 4, RP, LL)
    s1 = osr[0, 0, :NN, 0]
    s2 = osr[0, 1, :NN, 0]
    s3 = osr[0, 2, :NN, 0]
    s4 = osr[0, 3, :NN, 0]

    rd_sl = rowdeg - selfdeg + 1.0
    cd_sl = coldeg - selfdeg + 1.0
    douts = jnp.stack([rd_sl, cd_sl, s1, s3, s4, s2])
    dins = jnp.stack([cd_sl, rd_sl, s2, s3, s4, s1])

    def inv(d):
        return jnp.where(d == 0.0, 0.0, lax.rsqrt(d))

    y = _tc_matmul(x, W, inv(dins).T).reshape(6 * NC * RP, HALF)
    o1, o2 = _sc_prop(ei, y, z128)
    total = _tc_final(
        o1.reshape(6 * NC, RP, HALF),
        o2.reshape(4 * NC, RP, HALF),
        inv(douts).T,
        jnp.sum(b, axis=0, keepdims=True),
    )
    return total
